# baseline (speedup 1.0000x reference)
import functools

import jax
import jax.numpy as jnp
from jax import lax
from jax.experimental import pallas as pl
from jax.experimental.pallas import tpu as pltpu
from jax.experimental.pallas import tpu_sc as plsc

H = 512
W = 512
B = 16
N = 2_000_000
NBINS = 2 * H * W * B

NC = 2
NS = 16
NW = NC * NS

CH_ROWS = 1600
CH_IN = CH_ROWS * 5
VPC1 = CH_ROWS // 16
NCHUNK1 = N // CH_ROWS

WIN = 3 * 2 * H * W
PADWIN = WIN + 256
NWIN = 6
ZPT = PADWIN // NS
OPT = WIN // NS
CH2 = 4_000
NCHUNK2 = N // CH2
ZBUF = 4_096

_mesh = plsc.VectorSubcoreMesh(core_axis_name="c", subcore_axis_name="s")


@functools.partial(
    pl.kernel,
    out_type=jax.ShapeDtypeStruct((N,), jnp.int32),
    mesh=_mesh,
    scratch_types=[
        pltpu.VMEM((CH_IN,), jnp.float32),
        pltpu.VMEM((CH_ROWS,), jnp.int32),
    ],
    compiler_params=pltpu.CompilerParams(needs_layout_passes=False),
)
def _idx_kernel(ev_hbm, idx_hbm, ev_v, idx_v):
    wid = lax.axis_index("s") * NC + lax.axis_index("c")
    lane = lax.iota(jnp.int32, 16)
    lane5 = lane * 5
    nch = (NCHUNK1 - wid + NW - 1) // NW

    def chunk_body(i, _):
        cid = wid + i * NW
        pltpu.sync_copy(ev_hbm.at[pl.ds(cid * CH_IN, CH_IN)], ev_v)

        def vec_body(v, _):
            base = v * 80 + lane5
            x = plsc.load_gather(ev_v, [base]).astype(jnp.int32)
            y = plsc.load_gather(ev_v, [base + 1]).astype(jnp.int32)
            p = plsc.load_gather(ev_v, [base + 3]).astype(jnp.int32)
            b = plsc.load_gather(ev_v, [base + 4]).astype(jnp.int32)
            vi = x + y * W + p * (W * H) + b * (2 * W * H)
            idx_v[pl.ds(v * 16, 16)] = vi
            return 0

        lax.fori_loop(0, VPC1, vec_body, 0)
        pltpu.sync_copy(idx_v, idx_hbm.at[pl.ds(cid * CH_ROWS, CH_ROWS)])
        return 0

    lax.fori_loop(0, nch, chunk_body, 0)


@functools.partial(
    pl.kernel,
    out_type=jax.ShapeDtypeStruct((NBINS,), jnp.float32),
    mesh=_mesh,
    scratch_types=[
        pltpu.VMEM_SHARED((PADWIN,), jnp.float32),
        pltpu.VMEM((CH2,), jnp.int32),
        pltpu.VMEM((CH2,), jnp.int32),
        pltpu.VMEM((CH2,), jnp.float32),
        pltpu.VMEM((ZBUF,), jnp.float32),
    ],
    compiler_params=pltpu.CompilerParams(needs_layout_passes=False),
)
def _hist_kernel(idx_hbm, out_hbm, spmem, idx_v, rel_v, ones_v, zero_v):
    c = lax.axis_index("c")
    s = lax.axis_index("s")

    one16 = jnp.full((16,), 1.0, jnp.float32)
    zero16 = jnp.zeros((16,), jnp.float32)

    def fill_ones(r, _):
        ones_v[pl.ds(r * 16, 16)] = one16
        return 0

    lax.fori_loop(0, CH2 // 16, fill_ones, 0)

    def fill_zero(j, _):
        zero_v[pl.ds(j * 16, 16)] = zero16
        return 0

    lax.fori_loop(0, ZBUF // 16, fill_zero, 0)

    nch = (NCHUNK2 - s + NS - 1) // NS
    win_u = jnp.uint32(WIN)

    for p in range(3):
        w_lo = (2 * p + c) * WIN

        zoff = s * ZPT

        def zero_body(j, _):
            pltpu.sync_copy(zero_v, spmem.at[pl.ds(zoff + j * ZBUF, ZBUF)])
            return 0

        lax.fori_loop(0, ZPT // ZBUF, zero_body, 0)
        pltpu.sync_copy(zero_v.at[pl.ds(0, ZPT % ZBUF)],
                        spmem.at[pl.ds(zoff + (ZPT // ZBUF) * ZBUF, ZPT % ZBUF)])
        plsc.subcore_barrier()

        def chunk_body(i, _):
            cid = s + i * NS
            pltpu.sync_copy(idx_hbm.at[pl.ds(cid * CH2, CH2)], idx_v)

            def vec_body(v, _):
                iv = idx_v[pl.ds(v * 16, 16)]
                rel = plsc.bitcast(iv - w_lo, jnp.uint32)
                rel = jnp.minimum(rel, win_u)
                rel_v[pl.ds(v * 16, 16)] = plsc.bitcast(rel, jnp.int32)
                return 0

            lax.fori_loop(0, CH2 // 16, vec_body, 0)
            pltpu.sync_copy(ones_v, spmem.at[rel_v], add=True)
            return 0

        lax.fori_loop(0, nch, chunk_body, 0)
        plsc.subcore_barrier()

        if p < 2:
            off = s * OPT
            pltpu.sync_copy(spmem.at[pl.ds(off, OPT)],
                            out_hbm.at[pl.ds(w_lo + off, OPT)])
        else:
            @pl.when(c == 0)
            def _():
                off = s * OPT
                pltpu.sync_copy(spmem.at[pl.ds(off, OPT)],
                                out_hbm.at[pl.ds(w_lo + off, OPT)])

            last = (NBINS - 5 * WIN) // NS

            @pl.when(c == 1)
            def _():
                off = s * last
                pltpu.sync_copy(spmem.at[pl.ds(off, last)],
                                out_hbm.at[pl.ds(w_lo + off, last)])

        plsc.subcore_barrier()


@jax.jit
def kernel(events):
    ev_flat = events.reshape(-1)
    idx = _idx_kernel(ev_flat)
    vox = _hist_kernel(idx)
    return vox.reshape(-1, 2, H, W)


# speedup vs baseline: 3.5591x; 3.5591x over previous
import functools

import jax
import jax.numpy as jnp
from jax import lax
from jax.experimental import pallas as pl
from jax.experimental.pallas import tpu as pltpu
from jax.experimental.pallas import tpu_sc as plsc

H = 512
W = 512
B = 16
N = 2_000_000
NBINS = 2 * H * W * B

NC = 2
NS = 16
NW = NC * NS

CH_ROWS = 1600
CH_IN = CH_ROWS * 5
VPC1 = CH_ROWS // 16
NCHUNK1 = N // CH_ROWS

SLOT_BINS = 65_536
HPAD = SLOT_BINS + 16
SPB = 8
NPASS = (B * SPB) // NW
CH2 = 4_096
VPC2 = CH2 // 16

_mesh = plsc.VectorSubcoreMesh(core_axis_name="c", subcore_axis_name="s")


@functools.partial(
    pl.kernel,
    out_type=jax.ShapeDtypeStruct((N + CH2,), jnp.int32),
    mesh=_mesh,
    scratch_types=[
        pltpu.VMEM((CH_IN,), jnp.float32),
        pltpu.VMEM((CH_ROWS,), jnp.int32),
    ],
    compiler_params=pltpu.CompilerParams(needs_layout_passes=False),
)
def _idx_kernel(ev_hbm, idx_hbm, ev_v, idx_v):
    wid = lax.axis_index("s") * NC + lax.axis_index("c")
    lane5 = lax.iota(jnp.int32, 16) * 5
    nch = (NCHUNK1 - wid + NW - 1) // NW

    def chunk_body(i, _):
        cid = wid + i * NW
        pltpu.sync_copy(ev_hbm.at[pl.ds(cid * CH_IN, CH_IN)], ev_v)

        def vec_body(v, _):
            base = v * 80 + lane5
            x = plsc.load_gather(ev_v, [base]).astype(jnp.int32)
            y = plsc.load_gather(ev_v, [base + 1]).astype(jnp.int32)
            p = plsc.load_gather(ev_v, [base + 3]).astype(jnp.int32)
            b = plsc.load_gather(ev_v, [base + 4]).astype(jnp.int32)
            vi = x + y * W + p * (W * H) + b * (2 * W * H)
            idx_v[pl.ds(v * 16, 16)] = vi
            return 0

        lax.fori_loop(0, VPC1, vec_body, 0)
        pltpu.sync_copy(idx_v, idx_hbm.at[pl.ds(cid * CH_ROWS, CH_ROWS)])
        return 0

    lax.fori_loop(0, nch, chunk_body, 0)


@functools.partial(
    pl.kernel,
    out_type=jax.ShapeDtypeStruct((NBINS,), jnp.float32),
    mesh=_mesh,
    scratch_types=[
        pltpu.VMEM((HPAD,), jnp.float32),
        pltpu.VMEM((CH2,), jnp.int32),
        pltpu.VMEM((32,), jnp.int32),
    ],
    compiler_params=pltpu.CompilerParams(needs_layout_passes=False),
)
def _hist_kernel(idx_hbm, bnd_hbm, out_hbm, hist_v, idx_v, bnd_v):
    wid = lax.axis_index("s") * NC + lax.axis_index("c")
    pltpu.sync_copy(bnd_hbm, bnd_v)

    one16 = jnp.full((16,), 1.0, jnp.float32)
    zero16 = jnp.zeros((16,), jnp.float32)
    top = jnp.uint32(SLOT_BINS)

    for p in range(NPASS):
        slot = p * NW + wid
        beta = slot // SPB
        bin_base = slot * SLOT_BINS
        bnd_vec = bnd_v[pl.ds(beta, 16)]
        lo_e = bnd_vec[0]
        hi_e = bnd_vec[1]
        lo_v = lo_e // 16
        n_vec = (hi_e - lo_v * 16 + 15) // 16
        n_ch = (n_vec + VPC2 - 1) // VPC2

        def zero_body(j, _):
            hist_v[pl.ds(j * 16, 16)] = zero16
            return 0

        lax.fori_loop(0, HPAD // 16, zero_body, 0)

        def chunk_body(ci, _):
            base_e = lo_v * 16 + ci * CH2
            pltpu.sync_copy(idx_hbm.at[pl.ds(base_e, CH2)], idx_v)
            nv = jnp.minimum(n_vec - ci * VPC2, VPC2)

            def vec_body(v, _):
                iv = idx_v[pl.ds(v * 16, 16)]
                rel = plsc.bitcast(iv - bin_base, jnp.uint32)
                rel = jnp.minimum(rel, top)
                plsc.addupdate_scatter(
                    hist_v, [plsc.bitcast(rel, jnp.int32)], one16)
                return 0

            lax.fori_loop(0, nv, vec_body, 0)
            return 0

        lax.fori_loop(0, n_ch, chunk_body, 0)
        pltpu.sync_copy(hist_v.at[pl.ds(0, SLOT_BINS)],
                        out_hbm.at[pl.ds(bin_base, SLOT_BINS)])


@jax.jit
def kernel(events):
    ev_flat = events.reshape(-1)
    idx = _idx_kernel(ev_flat)
    bcol = events[:, 4]
    cuts = jnp.searchsorted(
        bcol, jnp.arange(1, B, dtype=bcol.dtype), side="left"
    ).astype(jnp.int32)
    bnd = jnp.concatenate([
        jnp.zeros((1,), jnp.int32),
        cuts,
        jnp.full((32 - B,), N, jnp.int32),
    ])
    vox = _hist_kernel(idx, bnd)
    return vox.reshape(-1, 2, H, W)


# speedup vs baseline: 4.1602x; 1.1689x over previous
import functools

import jax
import jax.numpy as jnp
from jax import lax
from jax.experimental import pallas as pl
from jax.experimental.pallas import tpu as pltpu
from jax.experimental.pallas import tpu_sc as plsc

H = 512
W = 512
B = 16
N = 2_000_000
NBINS = 2 * H * W * B

NC = 2
NS = 16
NW = NC * NS

CH_ROWS = 1600
CH_IN = CH_ROWS * 5
VPC1 = CH_ROWS // 16
NCHUNK1 = N // CH_ROWS

SLOT_BINS = 65_536
HPAD = SLOT_BINS + 16
SPB = 8
NPASS = (B * SPB) // NW
CH2 = 4_096
VPC2 = CH2 // 16

_mesh = plsc.VectorSubcoreMesh(core_axis_name="c", subcore_axis_name="s")


@functools.partial(
    pl.kernel,
    out_type=jax.ShapeDtypeStruct((N + CH2,), jnp.int32),
    mesh=_mesh,
    scratch_types=[
        pltpu.VMEM((CH_IN,), jnp.float32),
        pltpu.VMEM((CH_ROWS,), jnp.int32),
    ],
    compiler_params=pltpu.CompilerParams(needs_layout_passes=False),
)
def _idx_kernel(ev_hbm, idx_hbm, ev_v, idx_v):
    wid = lax.axis_index("s") * NC + lax.axis_index("c")
    lane5 = lax.iota(jnp.int32, 16) * 5
    nch = (NCHUNK1 - wid + NW - 1) // NW

    def chunk_body(i, _):
        cid = wid + i * NW
        pltpu.sync_copy(ev_hbm.at[pl.ds(cid * CH_IN, CH_IN)], ev_v)

        def vec_body(v, _):
            base = v * 80 + lane5
            x = plsc.load_gather(ev_v, [base]).astype(jnp.int32)
            y = plsc.load_gather(ev_v, [base + 1]).astype(jnp.int32)
            p = plsc.load_gather(ev_v, [base + 3]).astype(jnp.int32)
            b = plsc.load_gather(ev_v, [base + 4]).astype(jnp.int32)
            vi = x + y * W + p * (W * H) + b * (2 * W * H)
            idx_v[pl.ds(v * 16, 16)] = vi
            return 0

        lax.fori_loop(0, VPC1, vec_body, 0)
        pltpu.sync_copy(idx_v, idx_hbm.at[pl.ds(cid * CH_ROWS, CH_ROWS)])
        return 0

    lax.fori_loop(0, nch, chunk_body, 0)


@functools.partial(
    pl.kernel,
    out_type=jax.ShapeDtypeStruct((NBINS,), jnp.float32),
    mesh=_mesh,
    scratch_types=[
        pltpu.VMEM((HPAD,), jnp.float32),
        pltpu.VMEM((CH2,), jnp.int32),
        pltpu.VMEM((CH2,), jnp.int32),
        pltpu.VMEM((32,), jnp.int32),
        pltpu.SemaphoreType.DMA,
        pltpu.SemaphoreType.DMA,
    ],
    compiler_params=pltpu.CompilerParams(needs_layout_passes=False),
)
def _hist_kernel(idx_hbm, bnd_hbm, out_hbm, hist_v, idx_a, idx_b, bnd_v,
                 sem_a, sem_b):
    wid = lax.axis_index("s") * NC + lax.axis_index("c")
    pltpu.sync_copy(bnd_hbm, bnd_v)

    one16 = jnp.full((16,), 1.0, jnp.float32)
    zero16 = jnp.zeros((16,), jnp.float32)
    top = jnp.uint32(SLOT_BINS)

    for p in range(NPASS):
        slot = p * NW + wid
        beta = slot // SPB
        bin_base = slot * SLOT_BINS
        bnd_vec = bnd_v[pl.ds(beta, 16)]
        lo_e = bnd_vec[0]
        hi_e = bnd_vec[1]
        lo_v = lo_e // 16
        base0 = lo_v * 16
        n_vec = (hi_e - base0 + 15) // 16
        n_ch = (n_vec + VPC2 - 1) // VPC2

        @plsc.parallel_loop(0, HPAD // 16, unroll=8)
        def zero_body(j):
            hist_v[pl.ds(j * 16, 16)] = zero16

        def start(ci, buf, sem):
            b = jnp.minimum(base0 + ci * CH2, N)
            pltpu.async_copy(idx_hbm.at[pl.ds(b, CH2)], buf, sem)

        def drain(buf, sem):
            pltpu.make_async_copy(idx_hbm.at[pl.ds(0, CH2)], buf, sem).wait()

        def process(ci, buf):
            nv = jnp.clip(n_vec - ci * VPC2, 0, VPC2)

            @plsc.parallel_loop(0, nv, unroll=4)
            def vec_body(v):
                iv = buf[pl.ds(v * 16, 16)]
                rel = plsc.bitcast(iv - bin_base, jnp.uint32)
                rel = jnp.minimum(rel, top)
                plsc.addupdate_scatter(
                    hist_v, [plsc.bitcast(rel, jnp.int32)], one16)

        start(0, idx_a, sem_a)
        n_pair = (n_ch + 1) // 2

        def pair_body(g, _):
            c0 = 2 * g
            start(c0 + 1, idx_b, sem_b)
            drain(idx_a, sem_a)
            process(c0, idx_a)
            start(c0 + 2, idx_a, sem_a)
            drain(idx_b, sem_b)
            process(c0 + 1, idx_b)
            return 0

        lax.fori_loop(0, n_pair, pair_body, 0)
        drain(idx_a, sem_a)
        pltpu.sync_copy(hist_v.at[pl.ds(0, SLOT_BINS)],
                        out_hbm.at[pl.ds(bin_base, SLOT_BINS)])


@jax.jit
def kernel(events):
    ev_flat = events.reshape(-1)
    idx = _idx_kernel(ev_flat)
    bcol = events[:, 4]
    cuts = jnp.searchsorted(
        bcol, jnp.arange(1, B, dtype=bcol.dtype), side="left"
    ).astype(jnp.int32)
    bnd = jnp.concatenate([
        jnp.zeros((1,), jnp.int32),
        cuts,
        jnp.full((32 - B,), N, jnp.int32),
    ])
    vox = _hist_kernel(idx, bnd)
    return vox.reshape(-1, 2, H, W)


# speedup vs baseline: 4.1649x; 1.0011x over previous
import functools

import jax
import jax.numpy as jnp
from jax import lax
from jax.experimental import pallas as pl
from jax.experimental.pallas import tpu as pltpu
from jax.experimental.pallas import tpu_sc as plsc

H = 512
W = 512
B = 16
N = 2_000_000
NBINS = 2 * H * W * B

NC = 2
NS = 16
NW = NC * NS

CH_ROWS = 1600
CH_IN = CH_ROWS * 5
VPC1 = CH_ROWS // 16
NCHUNK1 = N // CH_ROWS

SLOT_BINS = 65_536
HPAD = SLOT_BINS + 16
SPB = 8
NPASS = (B * SPB) // NW
CH2 = 16_384
VPC2 = CH2 // 16

_mesh = plsc.VectorSubcoreMesh(core_axis_name="c", subcore_axis_name="s")


@functools.partial(
    pl.kernel,
    out_type=jax.ShapeDtypeStruct((N + CH2,), jnp.int32),
    mesh=_mesh,
    scratch_types=[
        pltpu.VMEM((CH_IN,), jnp.float32),
        pltpu.VMEM((CH_ROWS,), jnp.int32),
    ],
    compiler_params=pltpu.CompilerParams(needs_layout_passes=False),
)
def _idx_kernel(ev_hbm, idx_hbm, ev_v, idx_v):
    wid = lax.axis_index("s") * NC + lax.axis_index("c")
    lane5 = lax.iota(jnp.int32, 16) * 5
    nch = (NCHUNK1 - wid + NW - 1) // NW

    def chunk_body(i, _):
        cid = wid + i * NW
        pltpu.sync_copy(ev_hbm.at[pl.ds(cid * CH_IN, CH_IN)], ev_v)

        def vec_body(v, _):
            base = v * 80 + lane5
            x = plsc.load_gather(ev_v, [base]).astype(jnp.int32)
            y = plsc.load_gather(ev_v, [base + 1]).astype(jnp.int32)
            p = plsc.load_gather(ev_v, [base + 3]).astype(jnp.int32)
            b = plsc.load_gather(ev_v, [base + 4]).astype(jnp.int32)
            vi = x + y * W + p * (W * H) + b * (2 * W * H)
            idx_v[pl.ds(v * 16, 16)] = vi
            return 0

        lax.fori_loop(0, VPC1, vec_body, 0)
        pltpu.sync_copy(idx_v, idx_hbm.at[pl.ds(cid * CH_ROWS, CH_ROWS)])
        return 0

    lax.fori_loop(0, nch, chunk_body, 0)


@functools.partial(
    pl.kernel,
    out_type=jax.ShapeDtypeStruct((NBINS,), jnp.float32),
    mesh=_mesh,
    scratch_types=[
        pltpu.VMEM((HPAD,), jnp.float32),
        pltpu.VMEM((CH2,), jnp.int32),
        pltpu.VMEM((CH2,), jnp.int32),
        pltpu.VMEM((32,), jnp.int32),
        pltpu.SemaphoreType.DMA,
        pltpu.SemaphoreType.DMA,
    ],
    compiler_params=pltpu.CompilerParams(needs_layout_passes=False),
)
def _hist_kernel(idx_hbm, bnd_hbm, out_hbm, hist_v, idx_a, idx_b, bnd_v,
                 sem_a, sem_b):
    wid = lax.axis_index("s") * NC + lax.axis_index("c")
    pltpu.sync_copy(bnd_hbm, bnd_v)

    one16 = jnp.full((16,), 1.0, jnp.float32)
    zero16 = jnp.zeros((16,), jnp.float32)
    top = jnp.uint32(SLOT_BINS)

    for p in range(NPASS):
        slot = p * NW + wid
        beta = slot // SPB
        bin_base = slot * SLOT_BINS
        bnd_vec = bnd_v[pl.ds(beta, 16)]
        lo_e = bnd_vec[0]
        hi_e = bnd_vec[1]
        lo_v = lo_e // 16
        base0 = lo_v * 16
        n_vec = (hi_e - base0 + 15) // 16
        n_ch = (n_vec + VPC2 - 1) // VPC2

        @plsc.parallel_loop(0, HPAD // 16, unroll=8)
        def zero_body(j):
            hist_v[pl.ds(j * 16, 16)] = zero16

        def start(ci, buf, sem):
            b = jnp.minimum(base0 + ci * CH2, N)
            pltpu.async_copy(idx_hbm.at[pl.ds(b, CH2)], buf, sem)

        def drain(buf, sem):
            pltpu.make_async_copy(idx_hbm.at[pl.ds(0, CH2)], buf, sem).wait()

        def process(ci, buf):
            nv = jnp.clip(n_vec - ci * VPC2, 0, VPC2)

            @plsc.parallel_loop(0, nv, unroll=4)
            def vec_body(v):
                iv = buf[pl.ds(v * 16, 16)]
                rel = plsc.bitcast(iv - bin_base, jnp.uint32)
                rel = jnp.minimum(rel, top)
                plsc.addupdate_scatter(
                    hist_v, [plsc.bitcast(rel, jnp.int32)], one16)

        start(0, idx_a, sem_a)
        n_pair = (n_ch + 1) // 2

        def pair_body(g, _):
            c0 = 2 * g
            start(c0 + 1, idx_b, sem_b)
            drain(idx_a, sem_a)
            process(c0, idx_a)
            start(c0 + 2, idx_a, sem_a)
            drain(idx_b, sem_b)
            process(c0 + 1, idx_b)
            return 0

        lax.fori_loop(0, n_pair, pair_body, 0)
        drain(idx_a, sem_a)
        pltpu.sync_copy(hist_v.at[pl.ds(0, SLOT_BINS)],
                        out_hbm.at[pl.ds(bin_base, SLOT_BINS)])


@jax.jit
def kernel(events):
    ev_flat = events.reshape(-1)
    idx = _idx_kernel(ev_flat)
    bcol = events[:, 4]
    cuts = jnp.searchsorted(
        bcol, jnp.arange(1, B, dtype=bcol.dtype), side="left"
    ).astype(jnp.int32)
    bnd = jnp.concatenate([
        jnp.zeros((1,), jnp.int32),
        cuts,
        jnp.full((32 - B,), N, jnp.int32),
    ])
    vox = _hist_kernel(idx, bnd)
    return vox.reshape(-1, 2, H, W)


# speedup vs baseline: 4.5731x; 1.0980x over previous
import functools

import jax
import jax.numpy as jnp
from jax import lax
from jax.experimental import pallas as pl
from jax.experimental.pallas import tpu as pltpu
from jax.experimental.pallas import tpu_sc as plsc

H = 512
W = 512
B = 16
N = 2_000_000
NBINS = 2 * H * W * B

NC = 2
NS = 16
NW = NC * NS

CH_ROWS = 1600
CH_IN = CH_ROWS * 5
VPC1 = CH_ROWS // 16
NCHUNK1 = N // CH_ROWS

SLOT_BINS = 65_536
HPAD = SLOT_BINS + 16
SPB = 8
NPASS = (B * SPB) // NW
CH2 = 16_384
VPC2 = CH2 // 16

_mesh = plsc.VectorSubcoreMesh(core_axis_name="c", subcore_axis_name="s")


@functools.partial(
    pl.kernel,
    out_type=jax.ShapeDtypeStruct((N + CH2,), jnp.int32),
    mesh=_mesh,
    scratch_types=[
        pltpu.VMEM((CH_IN,), jnp.float32),
        pltpu.VMEM((CH_ROWS,), jnp.int32),
    ],
    compiler_params=pltpu.CompilerParams(needs_layout_passes=False),
)
def _idx_kernel(ev_hbm, idx_hbm, ev_v, idx_v):
    wid = lax.axis_index("s") * NC + lax.axis_index("c")
    lane5 = lax.iota(jnp.int32, 16) * 5
    nch = (NCHUNK1 - wid + NW - 1) // NW

    def chunk_body(i, _):
        cid = wid + i * NW
        pltpu.sync_copy(ev_hbm.at[pl.ds(cid * CH_IN, CH_IN)], ev_v)

        def vec_body(v, _):
            base = v * 80 + lane5
            x = plsc.load_gather(ev_v, [base]).astype(jnp.int32)
            y = plsc.load_gather(ev_v, [base + 1]).astype(jnp.int32)
            p = plsc.load_gather(ev_v, [base + 3]).astype(jnp.int32)
            b = plsc.load_gather(ev_v, [base + 4]).astype(jnp.int32)
            vi = x + y * W + p * (W * H) + b * (2 * W * H)
            idx_v[pl.ds(v * 16, 16)] = vi
            return 0

        lax.fori_loop(0, VPC1, vec_body, 0)
        pltpu.sync_copy(idx_v, idx_hbm.at[pl.ds(cid * CH_ROWS, CH_ROWS)])
        return 0

    lax.fori_loop(0, nch, chunk_body, 0)


@functools.partial(
    pl.kernel,
    out_type=jax.ShapeDtypeStruct((NBINS,), jnp.float32),
    mesh=_mesh,
    scratch_types=[
        pltpu.VMEM((HPAD,), jnp.float32),
        pltpu.VMEM((CH2,), jnp.int32),
        pltpu.VMEM((CH2,), jnp.int32),
        pltpu.VMEM((CH2 + 16,), jnp.int32),
        pltpu.VMEM((32,), jnp.int32),
        pltpu.SemaphoreType.DMA,
        pltpu.SemaphoreType.DMA,
    ],
    compiler_params=pltpu.CompilerParams(needs_layout_passes=False),
)
def _hist_kernel(idx_hbm, bnd_hbm, out_hbm, hist_v, idx_a, idx_b, comp_v,
                 bnd_v, sem_a, sem_b):
    wid = lax.axis_index("s") * NC + lax.axis_index("c")
    pltpu.sync_copy(bnd_hbm, bnd_v)

    one16 = jnp.full((16,), 1.0, jnp.float32)
    zero16 = jnp.zeros((16,), jnp.float32)
    sent16 = jnp.full((16,), SLOT_BINS, jnp.int32)
    top = jnp.uint32(SLOT_BINS)

    for p in range(NPASS):
        slot = p * NW + wid
        beta = slot // SPB
        bin_base = slot * SLOT_BINS
        bnd_vec = bnd_v[pl.ds(beta, 16)]
        lo_e = bnd_vec[0]
        hi_e = bnd_vec[1]
        lo_v = lo_e // 16
        base0 = lo_v * 16
        n_vec = (hi_e - base0 + 15) // 16
        n_ch = (n_vec + VPC2 - 1) // VPC2

        @plsc.parallel_loop(0, HPAD // 16, unroll=8)
        def zero_body(j):
            hist_v[pl.ds(j * 16, 16)] = zero16

        def start(ci, buf, sem):
            b = jnp.minimum(base0 + ci * CH2, N)
            pltpu.async_copy(idx_hbm.at[pl.ds(b, CH2)], buf, sem)

        def drain(buf, sem):
            pltpu.make_async_copy(idx_hbm.at[pl.ds(0, CH2)], buf, sem).wait()

        def process(ci, buf):
            nv = jnp.clip(n_vec - ci * VPC2, 0, VPC2)

            def p1_body(v, off):
                iv = buf[pl.ds(v * 16, 16)]
                rel = plsc.bitcast(iv - bin_base, jnp.uint32)
                m = rel < top
                plsc.store_compressed(comp_v.at[pl.ds(off, 16)],
                                      plsc.bitcast(rel, jnp.int32), mask=m)
                cnt = plsc.all_reduce_population_count(m)[0]
                return off + cnt

            off = lax.fori_loop(0, nv, p1_body, jnp.int32(0))
            comp_v[pl.ds(off, 16)] = sent16
            n2 = (off + 15) // 16

            @plsc.parallel_loop(0, n2, unroll=4)
            def p2_body(v):
                rv = comp_v[pl.ds(v * 16, 16)]
                plsc.addupdate_scatter(hist_v, [rv], one16)

        start(0, idx_a, sem_a)
        n_pair = (n_ch + 1) // 2

        def pair_body(g, _):
            c0 = 2 * g
            start(c0 + 1, idx_b, sem_b)
            drain(idx_a, sem_a)
            process(c0, idx_a)
            start(c0 + 2, idx_a, sem_a)
            drain(idx_b, sem_b)
            process(c0 + 1, idx_b)
            return 0

        lax.fori_loop(0, n_pair, pair_body, 0)
        drain(idx_a, sem_a)
        pltpu.sync_copy(hist_v.at[pl.ds(0, SLOT_BINS)],
                        out_hbm.at[pl.ds(bin_base, SLOT_BINS)])


@jax.jit
def kernel(events):
    ev_flat = events.reshape(-1)
    idx = _idx_kernel(ev_flat)
    bcol = events[:, 4]
    cuts = jnp.searchsorted(
        bcol, jnp.arange(1, B, dtype=bcol.dtype), side="left"
    ).astype(jnp.int32)
    bnd = jnp.concatenate([
        jnp.zeros((1,), jnp.int32),
        cuts,
        jnp.full((32 - B,), N, jnp.int32),
    ])
    vox = _hist_kernel(idx, bnd)
    return vox.reshape(-1, 2, H, W)
